# Initial kernel scaffold; baseline (speedup 1.0000x reference)
import jax
import jax.numpy as jnp
from jax.experimental import pallas as pl


def kernel(input_bat, node_feature, type_feature, length_feature, lane_feature, edge_index, struct_assign, fnc_assign, node_table, type_table, length_table, lane_table, fnc_gcn_W, fnc_gcn_b, struct_gcn_W, struct_gcn_b, gat_W, gat_b, lc_W, lc_b, ls_W, ls_b, bn_gamma, bn_beta, lin_W, lin_b):
    raise NotImplementedError("write your pallas kernel here")


# speedup vs baseline: 6.0666x; 6.0666x over previous
import functools

import jax
import jax.numpy as jnp
from jax import lax
from jax.experimental import pallas as pl
from jax.experimental.pallas import tpu as pltpu
from jax.experimental.pallas import tpu_sc as plsc

N = 10000
E = 320000
HID = 128
B = 4096
LABEL = 100

NC = 2
NT = 16
NW = NC * NT
EPW = E // NW
CHUNK = 80
NCHUNK = EPW // CHUNK
NP = 10240
RPT = NP // NT
BPW = B // NW

_MESH = plsc.VectorSubcoreMesh(core_axis_name="c", subcore_axis_name="s")


def _spmm_body(y_hbm, rows_hbm, cols_hbm, zeros_hbm, out_hbm,
               idx_row, idx_col, rows_v, acc, gsem):
    c = lax.axis_index("c")
    s = lax.axis_index("s")
    wid = c * NT + s
    pltpu.sync_copy(rows_hbm.at[wid], idx_row)
    pltpu.sync_copy(cols_hbm.at[wid], idx_col)
    pltpu.sync_copy(zeros_hbm.at[pl.ds(s * RPT, RPT)],
                    acc.at[pl.ds(s * RPT, RPT)])
    plsc.subcore_barrier()

    def step(j, carry):
        pltpu.async_copy(y_hbm.at[idx_col.at[j]], rows_v, gsem).wait()
        pltpu.sync_copy(rows_v, acc.at[idx_row.at[j]], add=True)
        return carry

    lax.fori_loop(0, NCHUNK, step, 0)
    plsc.subcore_barrier()
    pltpu.sync_copy(acc.at[pl.ds(s * RPT, RPT)],
                    out_hbm.at[c, pl.ds(s * RPT, RPT)])


_spmm_sc = functools.partial(
    pl.kernel,
    out_type=jax.ShapeDtypeStruct((NC, NP, HID), jnp.float32),
    mesh=_MESH,
    scratch_types=[
        pltpu.VMEM((NCHUNK, CHUNK), jnp.int32),
        pltpu.VMEM((NCHUNK, CHUNK), jnp.int32),
        pltpu.VMEM((CHUNK, HID), jnp.float32),
        pltpu.VMEM_SHARED((NP, HID), jnp.float32),
        pltpu.SemaphoreType.DMA,
    ],
)(_spmm_body)


def _gather_body(p0_hbm, p1_hbm, raw0_hbm, bat_hbm, out_hbm, idx_v, buf_v, gsem):
    wid = lax.axis_index("c") * NT + lax.axis_index("s")
    pltpu.sync_copy(bat_hbm.at[wid], idx_v)
    pltpu.async_copy(p0_hbm.at[idx_v], buf_v, gsem).wait()
    pltpu.sync_copy(buf_v, out_hbm.at[0, pl.ds(wid * BPW, BPW)])
    pltpu.async_copy(p1_hbm.at[idx_v], buf_v, gsem).wait()
    pltpu.sync_copy(buf_v, out_hbm.at[1, pl.ds(wid * BPW, BPW)])
    pltpu.async_copy(raw0_hbm.at[idx_v], buf_v, gsem).wait()
    pltpu.sync_copy(buf_v, out_hbm.at[2, pl.ds(wid * BPW, BPW)])


_gather_sc = functools.partial(
    pl.kernel,
    out_type=jax.ShapeDtypeStruct((3, B, HID), jnp.float32),
    mesh=_MESH,
    scratch_types=[
        pltpu.VMEM((BPW,), jnp.int32),
        pltpu.VMEM((BPW, HID), jnp.float32),
        pltpu.SemaphoreType.DMA,
    ],
)(_gather_body)


def _embed_tc(lane_f, type_f, len_f, t64, node_tab, gat_W, gat_b,
              raw0_ref, y1_ref):
    il = lax.broadcasted_iota(jnp.int32, (N, 128), 1)
    oh = ((il == lane_f[...]) | (il == type_f[...] + 8)
          | (il == len_f[...] + 28)).astype(jnp.float32)
    left = jnp.dot(oh, t64[...], preferred_element_type=jnp.float32)
    x = jnp.concatenate([left, node_tab[...]], axis=1)
    raw0_ref[...] = x
    y1_ref[...] = jnp.dot(x, gat_W[...],
                          preferred_element_type=jnp.float32) + gat_b[...]


def _embed_call(lane_f, type_f, len_f, t64, node_tab, gat_W, gat_b):
    return pl.pallas_call(
        _embed_tc,
        out_shape=(jax.ShapeDtypeStruct((N, HID), jnp.float32),
                   jax.ShapeDtypeStruct((N, HID), jnp.float32)),
    )(lane_f, type_f, len_f, t64, node_tab, gat_W, gat_b)


def _matmul_tc(a0, a1, W, b, y_ref):
    x = a0[...] + a1[...]
    y_ref[...] = jnp.dot(x, W[...], preferred_element_type=jnp.float32) + b[...]


def _matmul_call(a0, a1, W, b):
    return pl.pallas_call(
        _matmul_tc,
        out_shape=jax.ShapeDtypeStruct((N, HID), jnp.float32),
    )(a0, a1, W, b)


def _final_tc(g0, g1, gr, Wt, Wb, b, out_ref):
    x2 = g0[...] + g1[...]
    out_ref[...] = (jnp.dot(x2, Wt[...], preferred_element_type=jnp.float32)
                    + jnp.dot(gr[...], Wb[...],
                              preferred_element_type=jnp.float32) + b[...])


def _final_call(g0, g1, gr, Wt, Wb, b):
    return pl.pallas_call(
        _final_tc,
        out_shape=jax.ShapeDtypeStruct((B, LABEL), jnp.float32),
    )(g0, g1, gr, Wt, Wb, b)


def kernel(input_bat, node_feature, type_feature, length_feature, lane_feature,
           edge_index, struct_assign, fnc_assign, node_table, type_table,
           length_table, lane_table, fnc_gcn_W, fnc_gcn_b, struct_gcn_W,
           struct_gcn_b, gat_W, gat_b, lc_W, lc_b, ls_W, ls_b, bn_gamma,
           bn_beta, lin_W, lin_b):
    edge = edge_index.astype(jnp.int32)
    rows = edge[0].reshape(NW, NCHUNK, CHUNK)
    cols = edge[1].reshape(NW, NCHUNK, CHUNK)
    bat = input_bat.astype(jnp.int32).reshape(NW, BPW)
    zeros = jnp.zeros((NP, HID), jnp.float32)

    t64 = jnp.zeros((128, 64), jnp.float32)
    t64 = t64.at[0:8, 0:16].set(lane_table)
    t64 = t64.at[8:28, 16:48].set(type_table)
    t64 = t64.at[28:128, 48:64].set(length_table)

    lane_f = lane_feature.astype(jnp.int32).reshape(N, 1)
    type_f = type_feature.astype(jnp.int32).reshape(N, 1)
    len_f = length_feature.astype(jnp.int32).reshape(N, 1)

    gat_b2 = gat_b.reshape(1, HID)
    raw0, y1 = _embed_call(lane_f, type_f, len_f, t64, node_table, gat_W,
                           gat_b2)

    a = _spmm_sc(y1, rows, cols, zeros)
    y2 = _matmul_call(a[0, :N], a[1, :N], gat_W, gat_b2)
    p = _spmm_sc(y2, rows, cols, zeros)

    g = _gather_sc(p[0], p[1], raw0, bat)
    out = _final_call(g[0], g[1], g[2], lin_W[:HID], lin_W[HID:],
                      lin_b.reshape(1, LABEL))
    return out


# speedup vs baseline: 6.7944x; 1.1200x over previous
import functools

import jax
import jax.numpy as jnp
from jax import lax
from jax.experimental import pallas as pl
from jax.experimental.pallas import tpu as pltpu
from jax.experimental.pallas import tpu_sc as plsc

N = 10000
E = 320000
HID = 128
B = 4096
LABEL = 100

NC = 2
NT = 16
NW = NC * NT
CHUNK = 120
NCHUNK = 84
EPW = NCHUNK * CHUNK
NP = 10240
RPT = NP // NT
BPW = B // NW

_MESH = plsc.VectorSubcoreMesh(core_axis_name="c", subcore_axis_name="s")


def _spmm_body(y_hbm, rows_hbm, cols_hbm, zeros_hbm, out_hbm,
               idx_row, idx_col, rows_v, acc, gsem):
    c = lax.axis_index("c")
    s = lax.axis_index("s")
    wid = c * NT + s
    pltpu.sync_copy(rows_hbm.at[wid], idx_row)
    pltpu.sync_copy(cols_hbm.at[wid], idx_col)
    pltpu.sync_copy(zeros_hbm.at[pl.ds(s * RPT, RPT)],
                    acc.at[pl.ds(s * RPT, RPT)])
    plsc.subcore_barrier()

    def step(j, carry):
        pltpu.async_copy(y_hbm.at[idx_col.at[j]], rows_v, gsem).wait()
        pltpu.sync_copy(rows_v, acc.at[idx_row.at[j]], add=True)
        return carry

    lax.fori_loop(0, NCHUNK, step, 0)
    plsc.subcore_barrier()
    pltpu.sync_copy(acc.at[pl.ds(s * RPT, RPT)],
                    out_hbm.at[c, pl.ds(s * RPT, RPT)])


_spmm_sc = functools.partial(
    pl.kernel,
    out_type=jax.ShapeDtypeStruct((NC, NP, HID), jnp.float32),
    mesh=_MESH,
    scratch_types=[
        pltpu.VMEM((NCHUNK, CHUNK), jnp.int32),
        pltpu.VMEM((NCHUNK, CHUNK), jnp.int32),
        pltpu.VMEM((CHUNK, HID), jnp.float32),
        pltpu.VMEM_SHARED((NP, HID), jnp.float32),
        pltpu.SemaphoreType.DMA,
    ],
)(_spmm_body)


def _gather_body(p0_hbm, p1_hbm, raw0_hbm, bat_hbm, out_hbm, idx_v, buf_v, gsem):
    wid = lax.axis_index("c") * NT + lax.axis_index("s")
    pltpu.sync_copy(bat_hbm.at[wid], idx_v)
    pltpu.async_copy(p0_hbm.at[idx_v], buf_v, gsem).wait()
    pltpu.sync_copy(buf_v, out_hbm.at[0, pl.ds(wid * BPW, BPW)])
    pltpu.async_copy(p1_hbm.at[idx_v], buf_v, gsem).wait()
    pltpu.sync_copy(buf_v, out_hbm.at[1, pl.ds(wid * BPW, BPW)])
    pltpu.async_copy(raw0_hbm.at[idx_v], buf_v, gsem).wait()
    pltpu.sync_copy(buf_v, out_hbm.at[2, pl.ds(wid * BPW, BPW)])


_gather_sc = functools.partial(
    pl.kernel,
    out_type=jax.ShapeDtypeStruct((3, B, HID), jnp.float32),
    mesh=_MESH,
    scratch_types=[
        pltpu.VMEM((BPW,), jnp.int32),
        pltpu.VMEM((BPW, HID), jnp.float32),
        pltpu.SemaphoreType.DMA,
    ],
)(_gather_body)


def _embed_tc(lane_f, type_f, len_f, t64, node_tab, gat_W, gat_b,
              raw0_ref, y1_ref):
    il = lax.broadcasted_iota(jnp.int32, (N, 128), 1)
    oh = ((il == lane_f[...]) | (il == type_f[...] + 8)
          | (il == len_f[...] + 28)).astype(jnp.float32)
    left = jnp.dot(oh, t64[...], preferred_element_type=jnp.float32)
    x = jnp.concatenate([left, node_tab[...]], axis=1)
    raw0_ref[...] = x
    y1_ref[...] = jnp.dot(x, gat_W[...],
                          preferred_element_type=jnp.float32) + gat_b[...]


def _embed_call(lane_f, type_f, len_f, t64, node_tab, gat_W, gat_b):
    return pl.pallas_call(
        _embed_tc,
        out_shape=(jax.ShapeDtypeStruct((N, HID), jnp.float32),
                   jax.ShapeDtypeStruct((N, HID), jnp.float32)),
    )(lane_f, type_f, len_f, t64, node_tab, gat_W, gat_b)


def _matmul_tc(a0, a1, W, b, y_ref):
    x = a0[...] + a1[...]
    y_ref[...] = jnp.dot(x, W[...], preferred_element_type=jnp.float32) + b[...]


def _matmul_call(a0, a1, W, b):
    return pl.pallas_call(
        _matmul_tc,
        out_shape=jax.ShapeDtypeStruct((N, HID), jnp.float32),
    )(a0, a1, W, b)


def _final_tc(g0, g1, gr, Wt, Wb, b, out_ref):
    x2 = g0[...] + g1[...]
    out_ref[...] = (jnp.dot(x2, Wt[...], preferred_element_type=jnp.float32)
                    + jnp.dot(gr[...], Wb[...],
                              preferred_element_type=jnp.float32) + b[...])


def _final_call(g0, g1, gr, Wt, Wb, b):
    return pl.pallas_call(
        _final_tc,
        out_shape=jax.ShapeDtypeStruct((B, LABEL), jnp.float32),
    )(g0, g1, gr, Wt, Wb, b)


def kernel(input_bat, node_feature, type_feature, length_feature, lane_feature,
           edge_index, struct_assign, fnc_assign, node_table, type_table,
           length_table, lane_table, fnc_gcn_W, fnc_gcn_b, struct_gcn_W,
           struct_gcn_b, gat_W, gat_b, lc_W, lc_b, ls_W, ls_b, bn_gamma,
           bn_beta, lin_W, lin_b):
    edge = edge_index.astype(jnp.int32)
    pad_n = NW * EPW - E
    pad_r = N + jnp.arange(pad_n, dtype=jnp.int32) % (NP - N)
    pad_c = jnp.arange(pad_n, dtype=jnp.int32) % N
    rows = jnp.concatenate([edge[0], pad_r]).reshape(NW, NCHUNK, CHUNK)
    cols = jnp.concatenate([edge[1], pad_c]).reshape(NW, NCHUNK, CHUNK)
    bat = input_bat.astype(jnp.int32).reshape(NW, BPW)
    zeros = jnp.zeros((NP, HID), jnp.float32)

    t64 = jnp.zeros((128, 64), jnp.float32)
    t64 = t64.at[0:8, 0:16].set(lane_table)
    t64 = t64.at[8:28, 16:48].set(type_table)
    t64 = t64.at[28:128, 48:64].set(length_table)

    lane_f = lane_feature.astype(jnp.int32).reshape(N, 1)
    type_f = type_feature.astype(jnp.int32).reshape(N, 1)
    len_f = length_feature.astype(jnp.int32).reshape(N, 1)

    gat_b2 = gat_b.reshape(1, HID)
    raw0, y1 = _embed_call(lane_f, type_f, len_f, t64, node_table, gat_W,
                           gat_b2)

    a = _spmm_sc(y1, rows, cols, zeros)
    y2 = _matmul_call(a[0, :N], a[1, :N], gat_W, gat_b2)
    p = _spmm_sc(y2, rows, cols, zeros)

    g = _gather_sc(p[0], p[1], raw0, bat)
    out = _final_call(g[0], g[1], g[2], lin_W[:HID], lin_W[HID:],
                      lin_b.reshape(1, LABEL))
    return out


# speedup vs baseline: 6.8095x; 1.0022x over previous
import functools

import jax
import jax.numpy as jnp
from jax import lax
from jax.experimental import pallas as pl
from jax.experimental.pallas import tpu as pltpu
from jax.experimental.pallas import tpu_sc as plsc

N = 10000
E = 320000
HID = 128
B = 4096
LABEL = 100

NC = 2
NT = 16
NW = NC * NT
CHUNK = 120
NCHUNK = 84
EPW = NCHUNK * CHUNK
NP = 10240
RPT = NP // NT
BPW = B // NW

_MESH = plsc.VectorSubcoreMesh(core_axis_name="c", subcore_axis_name="s")


def _spmm_body(y_hbm, rows_hbm, cols_hbm, zeros_hbm, out_hbm,
               idx_row, idx_col, rows_v, acc, gsem):
    c = lax.axis_index("c")
    s = lax.axis_index("s")
    wid = c * NT + s
    pltpu.sync_copy(rows_hbm.at[wid], idx_row)
    pltpu.sync_copy(cols_hbm.at[wid], idx_col)
    pltpu.sync_copy(zeros_hbm.at[pl.ds(s * RPT, RPT)],
                    acc.at[pl.ds(s * RPT, RPT)])
    plsc.subcore_barrier()

    def step(j, carry):
        pltpu.async_copy(y_hbm.at[idx_col.at[j]], rows_v, gsem).wait()
        pltpu.sync_copy(rows_v, acc.at[idx_row.at[j]], add=True)
        return carry

    lax.fori_loop(0, NCHUNK, step, 0)
    plsc.subcore_barrier()
    pltpu.sync_copy(acc.at[pl.ds(s * RPT, RPT)],
                    out_hbm.at[c, pl.ds(s * RPT, RPT)])


_spmm_sc = functools.partial(
    pl.kernel,
    out_type=jax.ShapeDtypeStruct((NC, NP, HID), jnp.float32),
    mesh=_MESH,
    scratch_types=[
        pltpu.VMEM((NCHUNK, CHUNK), jnp.int32),
        pltpu.VMEM((NCHUNK, CHUNK), jnp.int32),
        pltpu.VMEM((CHUNK, HID), jnp.float32),
        pltpu.VMEM_SHARED((NP, HID), jnp.float32),
        pltpu.SemaphoreType.DMA,
    ],
)(_spmm_body)


def _gather_body(p0_hbm, p1_hbm, raw0_hbm, bat_hbm, out_hbm, idx_v, buf_v, gsem):
    wid = lax.axis_index("c") * NT + lax.axis_index("s")
    pltpu.sync_copy(bat_hbm.at[wid], idx_v)
    pltpu.async_copy(p0_hbm.at[idx_v], buf_v, gsem).wait()
    pltpu.sync_copy(buf_v, out_hbm.at[0, pl.ds(wid * BPW, BPW)])
    pltpu.async_copy(p1_hbm.at[idx_v], buf_v, gsem).wait()
    pltpu.sync_copy(buf_v, out_hbm.at[1, pl.ds(wid * BPW, BPW)])
    pltpu.async_copy(raw0_hbm.at[idx_v], buf_v, gsem).wait()
    pltpu.sync_copy(buf_v, out_hbm.at[2, pl.ds(wid * BPW, BPW)])


_gather_sc = functools.partial(
    pl.kernel,
    out_type=jax.ShapeDtypeStruct((3, B, HID), jnp.float32),
    mesh=_MESH,
    scratch_types=[
        pltpu.VMEM((BPW,), jnp.int32),
        pltpu.VMEM((BPW, HID), jnp.float32),
        pltpu.SemaphoreType.DMA,
    ],
)(_gather_body)


def _embed_tc(lane_f, type_f, len_f, t64, node_tab, gat_W, gat_b,
              raw0_ref, y1_ref):
    il = lax.broadcasted_iota(jnp.int32, (_EMB_BLK, 128), 1)
    oh = ((il == lane_f[...]) | (il == type_f[...] + 8)
          | (il == len_f[...] + 28)).astype(jnp.float32)
    left = jnp.dot(oh, t64[...], preferred_element_type=jnp.float32)
    x = jnp.concatenate([left, node_tab[...]], axis=1)
    raw0_ref[...] = x
    y1_ref[...] = jnp.dot(x, gat_W[...],
                          preferred_element_type=jnp.float32) + gat_b[...]


_EMB_BLK = 2000


def _embed_call(lane_f, type_f, len_f, t64, node_tab, gat_W, gat_b):
    return pl.pallas_call(
        _embed_tc,
        grid=(N // _EMB_BLK,),
        in_specs=[
            pl.BlockSpec((_EMB_BLK, 1), lambda i: (i, 0)),
            pl.BlockSpec((_EMB_BLK, 1), lambda i: (i, 0)),
            pl.BlockSpec((_EMB_BLK, 1), lambda i: (i, 0)),
            pl.BlockSpec((128, 64), lambda i: (0, 0)),
            pl.BlockSpec((_EMB_BLK, 64), lambda i: (i, 0)),
            pl.BlockSpec((HID, HID), lambda i: (0, 0)),
            pl.BlockSpec((1, HID), lambda i: (0, 0)),
        ],
        out_specs=(pl.BlockSpec((_EMB_BLK, HID), lambda i: (i, 0)),
                   pl.BlockSpec((_EMB_BLK, HID), lambda i: (i, 0))),
        out_shape=(jax.ShapeDtypeStruct((N, HID), jnp.float32),
                   jax.ShapeDtypeStruct((N, HID), jnp.float32)),
    )(lane_f, type_f, len_f, t64, node_tab, gat_W, gat_b)


def _matmul_tc(a0, a1, W, b, y_ref):
    x = a0[...] + a1[...]
    y_ref[...] = jnp.dot(x, W[...], preferred_element_type=jnp.float32) + b[...]


def _matmul_call(a0, a1, W, b):
    return pl.pallas_call(
        _matmul_tc,
        grid=(N // _EMB_BLK,),
        in_specs=[
            pl.BlockSpec((_EMB_BLK, HID), lambda i: (i, 0)),
            pl.BlockSpec((_EMB_BLK, HID), lambda i: (i, 0)),
            pl.BlockSpec((HID, HID), lambda i: (0, 0)),
            pl.BlockSpec((1, HID), lambda i: (0, 0)),
        ],
        out_specs=pl.BlockSpec((_EMB_BLK, HID), lambda i: (i, 0)),
        out_shape=jax.ShapeDtypeStruct((N, HID), jnp.float32),
    )(a0, a1, W, b)


def _final_tc(g0, g1, gr, Wt, Wb, b, out_ref):
    x2 = g0[...] + g1[...]
    out_ref[...] = (jnp.dot(x2, Wt[...], preferred_element_type=jnp.float32)
                    + jnp.dot(gr[...], Wb[...],
                              preferred_element_type=jnp.float32) + b[...])


_FIN_BLK = 1024


def _final_call(g0, g1, gr, Wt, Wb, b):
    return pl.pallas_call(
        _final_tc,
        grid=(B // _FIN_BLK,),
        in_specs=[
            pl.BlockSpec((_FIN_BLK, HID), lambda i: (i, 0)),
            pl.BlockSpec((_FIN_BLK, HID), lambda i: (i, 0)),
            pl.BlockSpec((_FIN_BLK, HID), lambda i: (i, 0)),
            pl.BlockSpec((HID, LABEL), lambda i: (0, 0)),
            pl.BlockSpec((HID, LABEL), lambda i: (0, 0)),
            pl.BlockSpec((1, LABEL), lambda i: (0, 0)),
        ],
        out_specs=pl.BlockSpec((_FIN_BLK, LABEL), lambda i: (i, 0)),
        out_shape=jax.ShapeDtypeStruct((B, LABEL), jnp.float32),
    )(g0, g1, gr, Wt, Wb, b)


def kernel(input_bat, node_feature, type_feature, length_feature, lane_feature,
           edge_index, struct_assign, fnc_assign, node_table, type_table,
           length_table, lane_table, fnc_gcn_W, fnc_gcn_b, struct_gcn_W,
           struct_gcn_b, gat_W, gat_b, lc_W, lc_b, ls_W, ls_b, bn_gamma,
           bn_beta, lin_W, lin_b):
    edge = edge_index.astype(jnp.int32)
    pad_n = NW * EPW - E
    pad_r = N + jnp.arange(pad_n, dtype=jnp.int32) % (NP - N)
    pad_c = jnp.arange(pad_n, dtype=jnp.int32) % N
    rows = jnp.concatenate([edge[0], pad_r]).reshape(NW, NCHUNK, CHUNK)
    cols = jnp.concatenate([edge[1], pad_c]).reshape(NW, NCHUNK, CHUNK)
    bat = input_bat.astype(jnp.int32).reshape(NW, BPW)
    zeros = jnp.zeros((NP, HID), jnp.float32)

    t64 = jnp.zeros((128, 64), jnp.float32)
    t64 = t64.at[0:8, 0:16].set(lane_table)
    t64 = t64.at[8:28, 16:48].set(type_table)
    t64 = t64.at[28:128, 48:64].set(length_table)

    lane_f = lane_feature.astype(jnp.int32).reshape(N, 1)
    type_f = type_feature.astype(jnp.int32).reshape(N, 1)
    len_f = length_feature.astype(jnp.int32).reshape(N, 1)

    gat_b2 = gat_b.reshape(1, HID)
    raw0, y1 = _embed_call(lane_f, type_f, len_f, t64, node_table, gat_W,
                           gat_b2)

    a = _spmm_sc(y1, rows, cols, zeros)
    y2 = _matmul_call(a[0, :N], a[1, :N], gat_W, gat_b2)
    p = _spmm_sc(y2, rows, cols, zeros)

    g = _gather_sc(p[0], p[1], raw0, bat)
    out = _final_call(g[0], g[1], g[2], lin_W[:HID], lin_W[HID:],
                      lin_b.reshape(1, LABEL))
    return out


# speedup vs baseline: 6.8708x; 1.0090x over previous
import functools

import jax
import jax.numpy as jnp
from jax import lax
from jax.experimental import pallas as pl
from jax.experimental.pallas import tpu as pltpu
from jax.experimental.pallas import tpu_sc as plsc

N = 10000
E = 320000
HID = 128
B = 4096
LABEL = 100

NC = 2
NT = 16
NW = NC * NT
CHUNK = 128
NCHUNK = 80
EPW = NCHUNK * CHUNK
NP = 10240
RPT = NP // NT
BPW = B // NW

_MESH = plsc.VectorSubcoreMesh(core_axis_name="c", subcore_axis_name="s")


def _spmm_body(y_hbm, rows_hbm, cols_hbm, zeros_hbm, out_hbm,
               idx_row, idx_col, rows_v, acc, gsem):
    c = lax.axis_index("c")
    s = lax.axis_index("s")
    wid = c * NT + s
    pltpu.sync_copy(rows_hbm.at[wid], idx_row)
    pltpu.sync_copy(cols_hbm.at[wid], idx_col)
    pltpu.sync_copy(zeros_hbm.at[pl.ds(s * RPT, RPT)],
                    acc.at[pl.ds(s * RPT, RPT)])
    plsc.subcore_barrier()

    def step(j, carry):
        pltpu.async_copy(y_hbm.at[idx_col.at[j]], rows_v, gsem).wait()
        pltpu.sync_copy(rows_v, acc.at[idx_row.at[j]], add=True)
        return carry

    lax.fori_loop(0, NCHUNK, step, 0)
    plsc.subcore_barrier()
    pltpu.sync_copy(acc.at[pl.ds(s * RPT, RPT)],
                    out_hbm.at[c, pl.ds(s * RPT, RPT)])


_spmm_sc = functools.partial(
    pl.kernel,
    out_type=jax.ShapeDtypeStruct((NC, NP, HID), jnp.float32),
    mesh=_MESH,
    scratch_types=[
        pltpu.VMEM((NCHUNK, CHUNK), jnp.int32),
        pltpu.VMEM((NCHUNK, CHUNK), jnp.int32),
        pltpu.VMEM((CHUNK, HID), jnp.float32),
        pltpu.VMEM_SHARED((NP, HID), jnp.float32),
        pltpu.SemaphoreType.DMA,
    ],
)(_spmm_body)


def _gather_body(p0_hbm, p1_hbm, raw0_hbm, bat_hbm, out_hbm, idx_v, buf_v, gsem):
    wid = lax.axis_index("c") * NT + lax.axis_index("s")
    pltpu.sync_copy(bat_hbm.at[wid], idx_v)
    pltpu.async_copy(p0_hbm.at[idx_v], buf_v, gsem).wait()
    pltpu.sync_copy(buf_v, out_hbm.at[0, pl.ds(wid * BPW, BPW)])
    pltpu.async_copy(p1_hbm.at[idx_v], buf_v, gsem).wait()
    pltpu.sync_copy(buf_v, out_hbm.at[1, pl.ds(wid * BPW, BPW)])
    pltpu.async_copy(raw0_hbm.at[idx_v], buf_v, gsem).wait()
    pltpu.sync_copy(buf_v, out_hbm.at[2, pl.ds(wid * BPW, BPW)])


_gather_sc = functools.partial(
    pl.kernel,
    out_type=jax.ShapeDtypeStruct((3, B, HID), jnp.float32),
    mesh=_MESH,
    scratch_types=[
        pltpu.VMEM((BPW,), jnp.int32),
        pltpu.VMEM((BPW, HID), jnp.float32),
        pltpu.SemaphoreType.DMA,
    ],
)(_gather_body)


def _embed_tc(lane_f, type_f, len_f, t64, node_tab, gat_W, gat_b,
              raw0_ref, y1_ref):
    il = lax.broadcasted_iota(jnp.int32, (_EMB_BLK, 128), 1)
    oh = ((il == lane_f[...]) | (il == type_f[...] + 8)
          | (il == len_f[...] + 28)).astype(jnp.float32)
    left = jnp.dot(oh, t64[...], preferred_element_type=jnp.float32)
    x = jnp.concatenate([left, node_tab[...]], axis=1)
    raw0_ref[...] = x
    y1_ref[...] = jnp.dot(x, gat_W[...],
                          preferred_element_type=jnp.float32) + gat_b[...]


_EMB_BLK = 2000


def _embed_call(lane_f, type_f, len_f, t64, node_tab, gat_W, gat_b):
    return pl.pallas_call(
        _embed_tc,
        grid=(N // _EMB_BLK,),
        in_specs=[
            pl.BlockSpec((_EMB_BLK, 1), lambda i: (i, 0)),
            pl.BlockSpec((_EMB_BLK, 1), lambda i: (i, 0)),
            pl.BlockSpec((_EMB_BLK, 1), lambda i: (i, 0)),
            pl.BlockSpec((128, 64), lambda i: (0, 0)),
            pl.BlockSpec((_EMB_BLK, 64), lambda i: (i, 0)),
            pl.BlockSpec((HID, HID), lambda i: (0, 0)),
            pl.BlockSpec((1, HID), lambda i: (0, 0)),
        ],
        out_specs=(pl.BlockSpec((_EMB_BLK, HID), lambda i: (i, 0)),
                   pl.BlockSpec((_EMB_BLK, HID), lambda i: (i, 0))),
        out_shape=(jax.ShapeDtypeStruct((N, HID), jnp.float32),
                   jax.ShapeDtypeStruct((N, HID), jnp.float32)),
    )(lane_f, type_f, len_f, t64, node_tab, gat_W, gat_b)


def _matmul_tc(a0, a1, W, b, y_ref):
    x = a0[...] + a1[...]
    y_ref[...] = jnp.dot(x, W[...], preferred_element_type=jnp.float32) + b[...]


def _matmul_call(a0, a1, W, b):
    return pl.pallas_call(
        _matmul_tc,
        grid=(N // _EMB_BLK,),
        in_specs=[
            pl.BlockSpec((_EMB_BLK, HID), lambda i: (i, 0)),
            pl.BlockSpec((_EMB_BLK, HID), lambda i: (i, 0)),
            pl.BlockSpec((HID, HID), lambda i: (0, 0)),
            pl.BlockSpec((1, HID), lambda i: (0, 0)),
        ],
        out_specs=pl.BlockSpec((_EMB_BLK, HID), lambda i: (i, 0)),
        out_shape=jax.ShapeDtypeStruct((N, HID), jnp.float32),
    )(a0, a1, W, b)


def _final_tc(g0, g1, gr, Wt, Wb, b, out_ref):
    x2 = g0[...] + g1[...]
    out_ref[...] = (jnp.dot(x2, Wt[...], preferred_element_type=jnp.float32)
                    + jnp.dot(gr[...], Wb[...],
                              preferred_element_type=jnp.float32) + b[...])


_FIN_BLK = 1024


def _final_call(g0, g1, gr, Wt, Wb, b):
    return pl.pallas_call(
        _final_tc,
        grid=(B // _FIN_BLK,),
        in_specs=[
            pl.BlockSpec((_FIN_BLK, HID), lambda i: (i, 0)),
            pl.BlockSpec((_FIN_BLK, HID), lambda i: (i, 0)),
            pl.BlockSpec((_FIN_BLK, HID), lambda i: (i, 0)),
            pl.BlockSpec((HID, LABEL), lambda i: (0, 0)),
            pl.BlockSpec((HID, LABEL), lambda i: (0, 0)),
            pl.BlockSpec((1, LABEL), lambda i: (0, 0)),
        ],
        out_specs=pl.BlockSpec((_FIN_BLK, LABEL), lambda i: (i, 0)),
        out_shape=jax.ShapeDtypeStruct((B, LABEL), jnp.float32),
    )(g0, g1, gr, Wt, Wb, b)


def kernel(input_bat, node_feature, type_feature, length_feature, lane_feature,
           edge_index, struct_assign, fnc_assign, node_table, type_table,
           length_table, lane_table, fnc_gcn_W, fnc_gcn_b, struct_gcn_W,
           struct_gcn_b, gat_W, gat_b, lc_W, lc_b, ls_W, ls_b, bn_gamma,
           bn_beta, lin_W, lin_b):
    edge = edge_index.astype(jnp.int32)
    pad_n = NW * EPW - E
    pad_r = N + jnp.arange(pad_n, dtype=jnp.int32) % (NP - N)
    pad_c = jnp.arange(pad_n, dtype=jnp.int32) % N
    rows = jnp.concatenate([edge[0], pad_r]).reshape(NW, NCHUNK, CHUNK)
    cols = jnp.concatenate([edge[1], pad_c]).reshape(NW, NCHUNK, CHUNK)
    bat = input_bat.astype(jnp.int32).reshape(NW, BPW)
    zeros = jnp.zeros((NP, HID), jnp.float32)

    t64 = jnp.zeros((128, 64), jnp.float32)
    t64 = t64.at[0:8, 0:16].set(lane_table)
    t64 = t64.at[8:28, 16:48].set(type_table)
    t64 = t64.at[28:128, 48:64].set(length_table)

    lane_f = lane_feature.astype(jnp.int32).reshape(N, 1)
    type_f = type_feature.astype(jnp.int32).reshape(N, 1)
    len_f = length_feature.astype(jnp.int32).reshape(N, 1)

    gat_b2 = gat_b.reshape(1, HID)
    raw0, y1 = _embed_call(lane_f, type_f, len_f, t64, node_table, gat_W,
                           gat_b2)

    a = _spmm_sc(y1, rows, cols, zeros)
    y2 = _matmul_call(a[0, :N], a[1, :N], gat_W, gat_b2)
    p = _spmm_sc(y2, rows, cols, zeros)

    g = _gather_sc(p[0], p[1], raw0, bat)
    out = _final_call(g[0], g[1], g[2], lin_W[:HID], lin_W[HID:],
                      lin_b.reshape(1, LABEL))
    return out
